# baseline (speedup 1.0000x reference)
import jax
import jax.numpy as jnp
from jax import lax
from jax.experimental import pallas as pl
from jax.experimental.pallas import tpu as pltpu
from jax.experimental.pallas import tpu_sc as plsc

D = 64
DR = 64
L = 20

_NC = 2
_NS = 16
_NW = _NC * _NS


def _wid():
    return lax.axis_index("s") * _NC + lax.axis_index("c")


def _bag_body(ids2, ew, out, idxv, rows0, rows1, bagst, gs0, gs1):
    wid = _wid()
    n_chunks = 32
    pltpu.sync_copy(ids2.at[pl.ds(wid * (n_chunks * 5), n_chunks * 5)], idxv)

    def start(chunk, rows, sem):
        for g in range(5):
            pltpu.async_copy(ew.at[idxv.at[chunk * 5 + g]],
                             rows.at[pl.ds(g * 128, 128)], sem)

    def drain(rows, sem):
        for g in range(5):
            pltpu.make_async_copy(ew.at[idxv.at[0]],
                                  rows.at[pl.ds(g * 128, 128)], sem).wait()

    def compute(chunk, rows):
        def el_body(el2, carry):
            for sub in range(4):
                el = el2 * 4 + sub
                r0 = el * L
                for q in range(4):
                    sl = pl.ds(q * 16, 16)
                    vals = [rows[r0 + t, sl] for t in range(L)]
                    while len(vals) > 1:
                        nxt = [vals[j] + vals[j + 1]
                               for j in range(0, len(vals) - 1, 2)]
                        if len(vals) % 2:
                            nxt.append(vals[-1])
                        vals = nxt
                    bagst[el, sl] = vals[0]
            return carry
        lax.fori_loop(0, 8, el_body, 0)
        pltpu.sync_copy(bagst, out.at[pl.ds(wid * (n_chunks * 32) + chunk * 32, 32)])

    start(0, rows0, gs0)
    start(1, rows1, gs1)

    def k_body(k, carry):
        c0 = 2 * k
        drain(rows0, gs0)
        compute(c0, rows0)

        @pl.when(c0 + 2 < n_chunks)
        def _():
            start(c0 + 2, rows0, gs0)

        drain(rows1, gs1)
        compute(c0 + 1, rows1)

        @pl.when(c0 + 3 < n_chunks)
        def _():
            start(c0 + 3, rows1, gs1)

        return carry

    lax.fori_loop(0, n_chunks // 2, k_body, 0)


def _lin_body(bag_ref, ids_ref, wt_ref, bt_ref, out_ref):
    freq = jnp.sum((ids_ref[...] != 0).astype(jnp.float32), axis=1, keepdims=True)
    x = bag_ref[...].astype(jnp.float32) * freq
    dn = (((1,), (1,)), ((), ()))
    y = lax.dot_general(x, wt_ref[...], dn, preferred_element_type=jnp.float32)
    out_ref[...] = jnp.tanh(y + bt_ref[...])


def _bil_body(stot, stot16, p2, er, out, pv, stb, otb16, rows0, rows1, outst,
              gs0, gs1):
    wid = _wid()
    B = out.shape[0] * 16
    ce = 8
    n_chunks = 64
    pltpu.sync_copy(p2.at[pl.ds(wid * n_chunks, n_chunks)], pv)
    pltpu.sync_copy(stot.at[pl.ds(wid * 512, 512)], stb)
    pltpu.sync_copy(stot16.at[pl.ds(B + wid * 512, 512)], otb16)

    def start(c, rows, sem):
        pltpu.async_copy(er.at[pv.at[c]], rows, sem)

    def drain(rows, sem):
        pltpu.make_async_copy(er.at[pv.at[0]], rows, sem).wait()

    lanes16 = lax.broadcasted_iota(jnp.int32, (16,), 0)

    def compute(c, rows, acc):
        def one_example(el, acc):
            ge = c * ce + el
            sv = [stb[ge, pl.ds(q * 16, 16)] for q in range(4)]
            uf = [jnp.zeros((16,), jnp.float32) for _ in range(4)]
            u = [jnp.zeros((32,), jnp.bfloat16) for _ in range(2)]
            for i in range(DR):
                lane = jnp.full((16,), i % 16, jnp.int32)
                sb = sv[i // 16].at[lane].get(mode="promise_in_bounds")
                sbp = plsc.pack(sb, sb, format=plsc.PackFormat.INTERLEAVED)
                for q in range(2):
                    u[q] = u[q] + sbp * rows[el, pl.ds(i * DR + q * 32, 32)]
                if i % 8 == 7:
                    for q in range(2):
                        ua, ub = plsc.unpack(u[q],
                                             format=plsc.PackFormat.INTERLEAVED)
                        uf[2 * q] = uf[2 * q] + ua
                        uf[2 * q + 1] = uf[2 * q + 1] + ub
                        u[q] = jnp.zeros((32,), jnp.bfloat16)
            t = jnp.zeros((16,), jnp.float32)
            for q in range(2):
                oa, ob = plsc.unpack(otb16[ge, pl.ds(q * 32, 32)],
                                     format=plsc.PackFormat.INTERLEAVED)
                t = t + uf[2 * q] * oa + uf[2 * q + 1] * ob
            pred = jnp.sum(t)
            return jnp.where(lanes16 == (ge % 16), pred, acc)

        def el_body(el2, acc):
            for sub in range(4):
                acc = one_example(el2 * 4 + sub, acc)
            return acc
        return lax.fori_loop(0, ce // 4, el_body, acc)

    start(0, rows0, gs0)
    start(1, rows1, gs1)

    def k_body(k, acc):
        c0 = 2 * k
        drain(rows0, gs0)
        acc = compute(c0, rows0, acc)

        @pl.when(c0 + 2 < n_chunks)
        def _():
            start(c0 + 2, rows0, gs0)

        drain(rows1, gs1)
        acc = compute(c0 + 1, rows1, acc)

        @pl.when(c0 + 3 < n_chunks)
        def _():
            start(c0 + 3, rows1, gs1)

        outst[k, :] = acc
        return acc

    lax.fori_loop(0, n_chunks // 2, k_body,
                  jnp.zeros((16,), jnp.float32))
    pltpu.sync_copy(outst, out.at[pl.ds(wid * 32, 32)])


def kernel(s, o, p, embed_words, embed_rel, W_t, b_t):
    B, seq = s.shape
    assert seq == L and B % (512 * _NW // 16) == 0

    mesh = plsc.VectorSubcoreMesh(core_axis_name="c", subcore_axis_name="s")

    ids = jnp.concatenate([s, o], axis=0)
    ids2 = ids.reshape(-1).reshape(2 * B * L // 128, 128)

    bag = pl.kernel(
        _bag_body,
        out_type=jax.ShapeDtypeStruct((2 * B, D), jnp.float32),
        mesh=mesh,
        scratch_types=[
            pltpu.VMEM((160, 128), jnp.int32),
            pltpu.VMEM((640, D), jnp.float32),
            pltpu.VMEM((640, D), jnp.float32),
            pltpu.VMEM((32, D), jnp.float32),
            pltpu.SemaphoreType.DMA,
            pltpu.SemaphoreType.DMA,
        ],
        compiler_params=pltpu.CompilerParams(use_tc_tiling_on_sc=False),
    )(ids2, embed_words)

    blk = 512
    stot = pl.pallas_call(
        _lin_body,
        grid=(2 * B // blk,),
        in_specs=[
            pl.BlockSpec((blk, D), lambda i: (i, 0)),
            pl.BlockSpec((blk, L), lambda i: (i, 0)),
            pl.BlockSpec((DR, D), lambda i: (0, 0)),
            pl.BlockSpec((1, DR), lambda i: (0, 0)),
        ],
        out_specs=pl.BlockSpec((blk, DR), lambda i: (i, 0)),
        out_shape=jax.ShapeDtypeStruct((2 * B, DR), jnp.float32),
    )(bag, ids, W_t, b_t.reshape(1, DR))

    p2 = p.reshape(B // 8, 8)
    stot16 = stot.astype(jnp.bfloat16)
    er16 = embed_rel.astype(jnp.bfloat16)
    pred = pl.kernel(
        _bil_body,
        out_type=jax.ShapeDtypeStruct((B // 16, 16), jnp.float32),
        mesh=mesh,
        scratch_types=[
            pltpu.VMEM((64, 8), jnp.int32),
            pltpu.VMEM((512, DR), jnp.float32),
            pltpu.VMEM((512, DR), jnp.bfloat16),
            pltpu.VMEM((8, DR * DR), jnp.bfloat16),
            pltpu.VMEM((8, DR * DR), jnp.bfloat16),
            pltpu.VMEM((32, 16), jnp.float32),
            pltpu.SemaphoreType.DMA,
            pltpu.SemaphoreType.DMA,
        ],
        compiler_params=pltpu.CompilerParams(use_tc_tiling_on_sc=False,
                                             needs_layout_passes=False),
    )(stot, stot16, p2, er16)

    return pred.reshape(B, 1)


# speedup vs baseline: 1.0193x; 1.0193x over previous
import jax
import jax.numpy as jnp
from jax import lax
from jax.experimental import pallas as pl
from jax.experimental.pallas import tpu as pltpu
from jax.experimental.pallas import tpu_sc as plsc

D = 64
DR = 64
L = 20

_NC = 2
_NS = 16
_NW = _NC * _NS


def _wid():
    return lax.axis_index("s") * _NC + lax.axis_index("c")


def _bag_body(ids2, ew, out, idxv, rows0, rows1, bagst, gs0, gs1):
    wid = _wid()
    n_chunks = 32
    pltpu.sync_copy(ids2.at[pl.ds(wid * (n_chunks * 5), n_chunks * 5)], idxv)

    def start(chunk, rows, sem):
        for g in range(5):
            pltpu.async_copy(ew.at[idxv.at[chunk * 5 + g]],
                             rows.at[pl.ds(g * 128, 128)], sem)

    def drain(rows, sem):
        for g in range(5):
            pltpu.make_async_copy(ew.at[idxv.at[0]],
                                  rows.at[pl.ds(g * 128, 128)], sem).wait()

    def compute(chunk, rows):
        def el_body(el2, carry):
            for sub in range(4):
                el = el2 * 4 + sub
                r0 = el * L
                for q in range(4):
                    sl = pl.ds(q * 16, 16)
                    vals = [rows[r0 + t, sl] for t in range(L)]
                    while len(vals) > 1:
                        nxt = [vals[j] + vals[j + 1]
                               for j in range(0, len(vals) - 1, 2)]
                        if len(vals) % 2:
                            nxt.append(vals[-1])
                        vals = nxt
                    bagst[el, sl] = vals[0]
            return carry
        lax.fori_loop(0, 8, el_body, 0)
        pltpu.sync_copy(bagst, out.at[pl.ds(wid * (n_chunks * 32) + chunk * 32, 32)])

    start(0, rows0, gs0)
    start(1, rows1, gs1)

    def k_body(k, carry):
        c0 = 2 * k
        drain(rows0, gs0)
        compute(c0, rows0)

        @pl.when(c0 + 2 < n_chunks)
        def _():
            start(c0 + 2, rows0, gs0)

        drain(rows1, gs1)
        compute(c0 + 1, rows1)

        @pl.when(c0 + 3 < n_chunks)
        def _():
            start(c0 + 3, rows1, gs1)

        return carry

    lax.fori_loop(0, n_chunks // 2, k_body, 0)


def _lin_body(bag_ref, ids_ref, wt_ref, bt_ref, out_ref, out16_ref):
    freq = jnp.sum((ids_ref[...] != 0).astype(jnp.float32), axis=1, keepdims=True)
    x = bag_ref[...].astype(jnp.float32) * freq
    dn = (((1,), (1,)), ((), ()))
    y = lax.dot_general(x, wt_ref[...], dn, preferred_element_type=jnp.float32)
    st = jnp.tanh(y + bt_ref[...])
    out_ref[...] = st
    out16_ref[...] = st.astype(jnp.bfloat16)


def _bil_body(stot, stot16, p2, er, out, pv, stb, otb16, rows0, rows1, outst,
              gs0, gs1):
    wid = _wid()
    B = out.shape[0] * 16
    ce = 8
    n_chunks = 64
    pltpu.sync_copy(p2.at[pl.ds(wid * n_chunks, n_chunks)], pv)
    pltpu.sync_copy(stot.at[pl.ds(wid * 512, 512)], stb)
    pltpu.sync_copy(stot16.at[pl.ds(B + wid * 512, 512)], otb16)

    def start(c, rows, sem):
        pltpu.async_copy(er.at[pv.at[c]], rows, sem)

    def drain(rows, sem):
        pltpu.make_async_copy(er.at[pv.at[0]], rows, sem).wait()

    lanes16 = lax.broadcasted_iota(jnp.int32, (16,), 0)

    def compute(c, rows, acc):
        def one_example(el, acc):
            ge = c * ce + el
            sv = [stb[ge, pl.ds(q * 16, 16)] for q in range(4)]
            uf = [jnp.zeros((16,), jnp.float32) for _ in range(4)]
            u = [jnp.zeros((32,), jnp.bfloat16) for _ in range(2)]
            for i in range(DR):
                lane = jnp.full((16,), i % 16, jnp.int32)
                sb = sv[i // 16].at[lane].get(mode="promise_in_bounds")
                sbp = plsc.pack(sb, sb, format=plsc.PackFormat.INTERLEAVED)
                for q in range(2):
                    u[q] = u[q] + sbp * rows[el, pl.ds(i * DR + q * 32, 32)]
                if i % 8 == 7:
                    for q in range(2):
                        ua, ub = plsc.unpack(u[q],
                                             format=plsc.PackFormat.INTERLEAVED)
                        uf[2 * q] = uf[2 * q] + ua
                        uf[2 * q + 1] = uf[2 * q + 1] + ub
                        u[q] = jnp.zeros((32,), jnp.bfloat16)
            t = jnp.zeros((16,), jnp.float32)
            for q in range(2):
                oa, ob = plsc.unpack(otb16[ge, pl.ds(q * 32, 32)],
                                     format=plsc.PackFormat.INTERLEAVED)
                t = t + uf[2 * q] * oa + uf[2 * q + 1] * ob
            pred = jnp.sum(t)
            return jnp.where(lanes16 == (ge % 16), pred, acc)

        def el_body(el2, acc):
            for sub in range(4):
                acc = one_example(el2 * 4 + sub, acc)
            return acc
        return lax.fori_loop(0, ce // 4, el_body, acc)

    start(0, rows0, gs0)
    start(1, rows1, gs1)

    def k_body(k, acc):
        c0 = 2 * k
        drain(rows0, gs0)
        acc = compute(c0, rows0, acc)

        @pl.when(c0 + 2 < n_chunks)
        def _():
            start(c0 + 2, rows0, gs0)

        drain(rows1, gs1)
        acc = compute(c0 + 1, rows1, acc)

        @pl.when(c0 + 3 < n_chunks)
        def _():
            start(c0 + 3, rows1, gs1)

        outst[k, :] = acc
        return acc

    lax.fori_loop(0, n_chunks // 2, k_body,
                  jnp.zeros((16,), jnp.float32))
    pltpu.sync_copy(outst, out.at[pl.ds(wid * 32, 32)])


def kernel(s, o, p, embed_words, embed_rel, W_t, b_t):
    B, seq = s.shape
    assert seq == L and B % (512 * _NW // 16) == 0

    mesh = plsc.VectorSubcoreMesh(core_axis_name="c", subcore_axis_name="s")

    ids = jnp.concatenate([s, o], axis=0)
    ids2 = ids.reshape(-1).reshape(2 * B * L // 128, 128)

    bag = pl.kernel(
        _bag_body,
        out_type=jax.ShapeDtypeStruct((2 * B, D), jnp.float32),
        mesh=mesh,
        scratch_types=[
            pltpu.VMEM((160, 128), jnp.int32),
            pltpu.VMEM((640, D), jnp.float32),
            pltpu.VMEM((640, D), jnp.float32),
            pltpu.VMEM((32, D), jnp.float32),
            pltpu.SemaphoreType.DMA,
            pltpu.SemaphoreType.DMA,
        ],
        compiler_params=pltpu.CompilerParams(use_tc_tiling_on_sc=False),
    )(ids2, embed_words)

    blk = 512
    stot, stot16 = pl.pallas_call(
        _lin_body,
        grid=(2 * B // blk,),
        in_specs=[
            pl.BlockSpec((blk, D), lambda i: (i, 0)),
            pl.BlockSpec((blk, L), lambda i: (i, 0)),
            pl.BlockSpec((DR, D), lambda i: (0, 0)),
            pl.BlockSpec((1, DR), lambda i: (0, 0)),
        ],
        out_specs=[
            pl.BlockSpec((blk, DR), lambda i: (i, 0)),
            pl.BlockSpec((blk, DR), lambda i: (i, 0)),
        ],
        out_shape=[
            jax.ShapeDtypeStruct((2 * B, DR), jnp.float32),
            jax.ShapeDtypeStruct((2 * B, DR), jnp.bfloat16),
        ],
    )(bag, ids, W_t, b_t.reshape(1, DR))

    p2 = p.reshape(B // 8, 8)
    er16 = embed_rel.astype(jnp.bfloat16)
    pred = pl.kernel(
        _bil_body,
        out_type=jax.ShapeDtypeStruct((B // 16, 16), jnp.float32),
        mesh=mesh,
        scratch_types=[
            pltpu.VMEM((64, 8), jnp.int32),
            pltpu.VMEM((512, DR), jnp.float32),
            pltpu.VMEM((512, DR), jnp.bfloat16),
            pltpu.VMEM((8, DR * DR), jnp.bfloat16),
            pltpu.VMEM((8, DR * DR), jnp.bfloat16),
            pltpu.VMEM((32, 16), jnp.float32),
            pltpu.SemaphoreType.DMA,
            pltpu.SemaphoreType.DMA,
        ],
        compiler_params=pltpu.CompilerParams(use_tc_tiling_on_sc=False,
                                             needs_layout_passes=False),
    )(stot, stot16, p2, er16)

    return pred.reshape(B, 1)
